# Initial kernel scaffold; baseline (speedup 1.0000x reference)
import jax
import jax.numpy as jnp
from jax.experimental import pallas as pl


def kernel(sequence, segment, token_table, seg_table, pe):
    raise NotImplementedError("write your pallas kernel here")


# speedup vs baseline: 5.2941x; 5.2941x over previous
import functools
import math

import jax
import jax.numpy as jnp
from jax import lax
from jax.experimental import pallas as pl
from jax.experimental.pallas import tpu as pltpu
from jax.experimental.pallas import tpu_sc as plsc


def _segpe_table(seg_table, pe2d):
    seg_n, d = seg_table.shape
    l_n = pe2d.shape[0]

    def body(seg_ref, pe_ref, out_ref):
        out_ref[...] = seg_ref[...][:, None, :] + pe_ref[...][None, :, :]

    return pl.pallas_call(
        body,
        out_shape=jax.ShapeDtypeStruct((seg_n, l_n, d), jnp.float32),
    )(seg_table, pe2d)


def _make_sc_gather(n_rows, d, l_n, chunk):
    info = plsc.get_sparse_core_info()
    nw = info.num_cores * info.num_subcores
    lanes = info.num_lanes
    assert n_rows % (nw * chunk) == 0
    per_w = n_rows // nw
    n_chunks = per_w // chunk
    mesh = plsc.VectorSubcoreMesh(core_axis_name="c", subcore_axis_name="s")

    @functools.partial(
        pl.kernel,
        mesh=mesh,
        out_type=jax.ShapeDtypeStruct((n_rows, d), jnp.float32),
        scratch_types=[
            pltpu.VMEM((chunk,), jnp.int32),
            pltpu.VMEM((chunk,), jnp.int32),
            pltpu.VMEM((chunk,), jnp.int32),
            pltpu.VMEM((chunk, d), jnp.float32),
            pltpu.VMEM((chunk, d), jnp.float32),
            pltpu.SemaphoreType.DMA,
            pltpu.SemaphoreType.DMA,
        ],
    )
    def sc_kernel(seq_hbm, seg_hbm, tok_hbm, segpe_hbm, out_hbm,
                  seqi_v, segi_v, idx2_v, tok_v, spe_v, sem_a, sem_b):
        wid = lax.axis_index("s") * info.num_cores + lax.axis_index("c")
        base = wid * per_w

        def chunk_body(g, carry):
            cb = base + g * chunk
            pltpu.sync_copy(seq_hbm.at[pl.ds(cb, chunk)], seqi_v)
            pltpu.sync_copy(seg_hbm.at[pl.ds(cb, chunk)], segi_v)
            for k in range(chunk // lanes):
                seg_vec = segi_v[pl.ds(k * lanes, lanes)]
                flat = (cb + k * lanes) + lax.iota(jnp.int32, lanes)
                idx2_v[pl.ds(k * lanes, lanes)] = seg_vec * l_n + flat % l_n
            cp_a = pltpu.async_copy(tok_hbm.at[seqi_v], tok_v, sem_a)
            cp_b = pltpu.async_copy(segpe_hbm.at[idx2_v], spe_v, sem_b)
            cp_a.wait()
            cp_b.wait()

            def add_body(r, c2):
                for c in range(d // lanes):
                    sl = pl.ds(c * lanes, lanes)
                    tok_v[r, sl] = tok_v[r, sl] + spe_v[r, sl]
                return c2

            lax.fori_loop(0, chunk, add_body, 0)
            pltpu.sync_copy(tok_v, out_hbm.at[pl.ds(cb, chunk)])
            return carry

        lax.fori_loop(0, n_chunks, chunk_body, 0)

    return sc_kernel


def kernel(sequence, segment, token_table, seg_table, pe):
    b, l_n = sequence.shape
    d = token_table.shape[1]
    n_rows = b * l_n

    seq_flat = sequence.reshape(n_rows).astype(jnp.int32)
    seg_flat = segment.reshape(n_rows).astype(jnp.int32)
    segpe = _segpe_table(seg_table, pe[0, :l_n]).reshape(-1, d)

    sc = _make_sc_gather(n_rows, d, l_n, chunk=128)
    out_flat = sc(seq_flat, seg_flat, token_table, segpe)
    return out_flat.reshape(b, l_n, d)


# speedup vs baseline: 6.8734x; 1.2983x over previous
import functools

import jax
import jax.numpy as jnp
from jax import lax
from jax.experimental import pallas as pl
from jax.experimental.pallas import tpu as pltpu
from jax.experimental.pallas import tpu_sc as plsc


def _segpe_table(seg_table, pe2d):
    seg_n, d = seg_table.shape
    l_n = pe2d.shape[0]

    def body(seg_ref, pe_ref, out_ref):
        out_ref[...] = seg_ref[...][:, None, :] + pe_ref[...][None, :, :]

    return pl.pallas_call(
        body,
        out_shape=jax.ShapeDtypeStruct((seg_n, l_n, d), jnp.float32),
    )(seg_table, pe2d)


def _make_sc_gather(n_rows, d, l_n, chunk):
    info = plsc.get_sparse_core_info()
    nw = info.num_cores * info.num_subcores
    lanes = info.num_lanes
    assert n_rows % (nw * chunk) == 0
    per_w = n_rows // nw
    n_chunks = per_w // chunk
    assert n_chunks % 2 == 0 and chunk <= 128
    mesh = plsc.VectorSubcoreMesh(core_axis_name="c", subcore_axis_name="s")

    @functools.partial(
        pl.kernel,
        mesh=mesh,
        out_type=jax.ShapeDtypeStruct((n_rows, d), jnp.float32),
        scratch_types=[
            pltpu.VMEM((n_chunks, chunk), jnp.int32),
            pltpu.VMEM((n_chunks, chunk), jnp.int32),
            pltpu.VMEM((chunk, d), jnp.float32),
            pltpu.VMEM((chunk, d), jnp.float32),
            pltpu.VMEM((chunk, d), jnp.float32),
            pltpu.VMEM((chunk, d), jnp.float32),
            pltpu.VMEM((chunk, d), jnp.float32),
            pltpu.VMEM((chunk, d), jnp.float32),
            pltpu.SemaphoreType.DMA,
            pltpu.SemaphoreType.DMA,
            pltpu.SemaphoreType.DMA,
            pltpu.SemaphoreType.DMA,
            pltpu.SemaphoreType.DMA,
            pltpu.SemaphoreType.DMA,
        ],
    )
    def sc_kernel(seq_hbm, seg_hbm, tok_hbm, segpe_hbm, out_hbm,
                  seqi_v, idx2_v, tok_a, tok_b, spe_a, spe_b, oub_a, oub_b,
                  st_a, st_b, ss_a, ss_b, so_a, so_b):
        wid = lax.axis_index("s") * info.num_cores + lax.axis_index("c")
        base = wid * per_w
        tok_bufs, spe_bufs, out_bufs = (tok_a, tok_b), (spe_a, spe_b), (oub_a, oub_b)
        sems_t, sems_s, sems_o = (st_a, st_b), (ss_a, ss_b), (so_a, so_b)

        pltpu.sync_copy(seq_hbm.at[wid], seqi_v)
        pltpu.sync_copy(seg_hbm.at[wid], idx2_v)

        def idx_body(i, carry):
            for j in range(chunk // lanes):
                sl = pl.ds(j * lanes, lanes)
                flat = (base + i * chunk + j * lanes) + lax.iota(jnp.int32, lanes)
                idx2_v[i, sl] = idx2_v[i, sl] * l_n + flat % l_n
            return carry

        lax.fori_loop(0, n_chunks, idx_body, 0)

        def tok_dma(g, b):
            return pltpu.make_async_copy(tok_hbm.at[seqi_v.at[g]], tok_bufs[b], sems_t[b])

        def spe_dma(g, b):
            return pltpu.make_async_copy(segpe_hbm.at[idx2_v.at[g]], spe_bufs[b], sems_s[b])

        def out_dma(g, b):
            return pltpu.make_async_copy(
                out_bufs[b], out_hbm.at[pl.ds(base + g * chunk, chunk)], sems_o[b])

        def iter_body(t, carry):
            for b in range(2):
                g = 2 * t + b
                tok_dma(g, b).start()
                spe_dma(g, b).start()
                tok_dma(g, b).wait()
                spe_dma(g, b).wait()

                @pl.when(g >= 2)
                def _drain_out():
                    out_dma(g - 2, b).wait()

                def add_body(r, c2):
                    for c in range(d // lanes):
                        sl = pl.ds(c * lanes, lanes)
                        out_bufs[b][r, sl] = tok_bufs[b][r, sl] + spe_bufs[b][r, sl]
                    return c2

                lax.fori_loop(0, chunk, add_body, 0)
                out_dma(g, b).start()
            return carry

        lax.fori_loop(0, n_chunks // 2, iter_body, 0)
        out_dma(n_chunks - 2, 0).wait()
        out_dma(n_chunks - 1, 1).wait()

    return sc_kernel


def kernel(sequence, segment, token_table, seg_table, pe):
    b, l_n = sequence.shape
    d = token_table.shape[1]
    n_rows = b * l_n
    chunk = 128

    nw = 32
    seq2d = sequence.reshape(nw, n_rows // (nw * chunk), chunk).astype(jnp.int32)
    seg2d = segment.reshape(nw, n_rows // (nw * chunk), chunk).astype(jnp.int32)
    segpe = _segpe_table(seg_table, pe[0, :l_n]).reshape(-1, d)

    sc = _make_sc_gather(n_rows, d, l_n, chunk)
    out_flat = sc(seq2d, seg2d, token_table, segpe)
    return out_flat.reshape(b, l_n, d)
